# Initial kernel scaffold; baseline (speedup 1.0000x reference)
import jax
import jax.numpy as jnp
from jax.experimental import pallas as pl


def kernel(port_idx, protocol_idx, features, port_table, proto_table, W1, b1, W2, b2, W3, b3):
    raise NotImplementedError("write your pallas kernel here")


# speedup vs baseline: 1.3693x; 1.3693x over previous
import functools

import jax
import jax.numpy as jnp
from jax import lax
from jax.experimental import pallas as pl
from jax.experimental.pallas import tpu as pltpu
from jax.experimental.pallas import tpu_sc as plsc

PORT_DIM = 32
PROTO_PAD = 16
CHUNK = 128


def _sc_gather(port_table, pidx, proto_table_p, qidx, *, nc, ns, b):
    nw = nc * ns
    bpw = b // nw
    nck = bpw // CHUNK
    mesh = plsc.VectorSubcoreMesh(core_axis_name="c", subcore_axis_name="s")

    @functools.partial(
        pl.kernel,
        mesh=mesh,
        compiler_params=pltpu.CompilerParams(use_tc_tiling_on_sc=False),
        out_type=[
            jax.ShapeDtypeStruct((b, PORT_DIM), jnp.float32),
            jax.ShapeDtypeStruct((b, PROTO_PAD), jnp.float32),
        ],
        scratch_types=[
            pltpu.VMEM((nck, CHUNK), jnp.int32),
            pltpu.VMEM((bpw, PORT_DIM), jnp.float32),
            pltpu.VMEM((nck, CHUNK), jnp.int32),
            pltpu.VMEM((bpw, PROTO_PAD), jnp.float32),
            pltpu.SemaphoreType.DMA,
            pltpu.SemaphoreType.DMA,
        ],
    )
    def gather(ptbl, pidx_hbm, qtbl, qidx_hbm, pout, qout,
               pidx_v, prow_v, qidx_v, qrow_v, psem, qsem):
        wid = lax.axis_index("s") * nc + lax.axis_index("c")
        base = wid * bpw
        pltpu.sync_copy(pidx_hbm.at[wid], pidx_v)
        pltpu.sync_copy(qidx_hbm.at[wid], qidx_v)
        copies = []
        for j in range(nck):
            copies.append(pltpu.async_copy(
                ptbl.at[pidx_v.at[j]], prow_v.at[pl.ds(j * CHUNK, CHUNK)], psem))
            copies.append(pltpu.async_copy(
                qtbl.at[qidx_v.at[j]], qrow_v.at[pl.ds(j * CHUNK, CHUNK)], qsem))
        for c in copies:
            c.wait()
        pltpu.sync_copy(prow_v, pout.at[pl.ds(base, bpw)])
        pltpu.sync_copy(qrow_v, qout.at[pl.ds(base, bpw)])

    return gather(port_table, pidx, proto_table_p, qidx)


def _mlp_body(port, proto, feat, w1a, w1b, w1c, b1, w2, b2, w3, b3, out):
    h = jnp.dot(feat[...], w1c[...], preferred_element_type=jnp.float32)
    h += jnp.dot(port[...], w1a[...], preferred_element_type=jnp.float32)
    h += jnp.dot(proto[...], w1b[...], preferred_element_type=jnp.float32)
    h = jnp.maximum(h + b1[...], 0.0)
    h = jnp.dot(h, w2[...], preferred_element_type=jnp.float32) + b2[...]
    h = jnp.maximum(h, 0.0)
    out[...] = jnp.dot(h, w3[...], preferred_element_type=jnp.float32) + b3[...]


def _mlp(port_emb, proto_emb, features, w1a, w1b, w1c, b1, w2, b2, w3, b3,
         *, bm):
    b = features.shape[0]
    n_out = w3.shape[1]
    const = lambda i: (0, 0)
    row = lambda i: (i, 0)
    return pl.pallas_call(
        _mlp_body,
        grid=(b // bm,),
        in_specs=[
            pl.BlockSpec((bm, PORT_DIM), row),
            pl.BlockSpec((bm, PROTO_PAD), row),
            pl.BlockSpec((bm, features.shape[1]), row),
            pl.BlockSpec(w1a.shape, const),
            pl.BlockSpec(w1b.shape, const),
            pl.BlockSpec(w1c.shape, const),
            pl.BlockSpec(b1.shape, const),
            pl.BlockSpec(w2.shape, const),
            pl.BlockSpec(b2.shape, const),
            pl.BlockSpec(w3.shape, const),
            pl.BlockSpec(b3.shape, const),
        ],
        out_specs=pl.BlockSpec((bm, n_out), row),
        out_shape=jax.ShapeDtypeStruct((b, n_out), jnp.float32),
    )(port_emb, proto_emb, features, w1a, w1b, w1c, b1, w2, b2, w3, b3)


def kernel(port_idx, protocol_idx, features, port_table, proto_table,
           W1, b1, W2, b2, W3, b3):
    b = port_idx.shape[0]
    info = plsc.get_sparse_core_info()
    nc, ns = info.num_cores, info.num_subcores
    nw = nc * ns

    pidx = port_idx.astype(jnp.int32).reshape(nw, -1, CHUNK)
    qidx = protocol_idx.astype(jnp.int32).reshape(nw, -1, CHUNK)
    proto_table_p = jnp.pad(proto_table,
                            ((0, 0), (0, PROTO_PAD - proto_table.shape[1])))

    port_emb, proto_emb = _sc_gather(port_table, pidx, proto_table_p, qidx,
                                     nc=nc, ns=ns, b=b)

    feat_dim = features.shape[1]
    w1a = W1[:PORT_DIM]
    w1b = jnp.pad(W1[PORT_DIM:PORT_DIM + proto_table.shape[1]],
                  ((0, PROTO_PAD - proto_table.shape[1]), (0, 0)))
    w1c = W1[PORT_DIM + proto_table.shape[1]:]
    return _mlp(port_emb, proto_emb, features,
                w1a, w1b, w1c, b1.reshape(1, -1),
                W2, b2.reshape(1, -1), W3, b3.reshape(1, -1), bm=512)


# speedup vs baseline: 1.9722x; 1.4403x over previous
import functools

import jax
import jax.numpy as jnp
from jax import lax
from jax.experimental import pallas as pl
from jax.experimental.pallas import tpu as pltpu
from jax.experimental.pallas import tpu_sc as plsc

PORT_DIM = 32
PROTO_DIM = 8
PACK_DIM = PORT_DIM + PROTO_DIM
LANES = 16


def _sc_gather_t(table_t, pidx, ptable_t, qidx, *, nc, ns, b):
    nw = nc * ns
    v = table_t.shape[1]
    pv = ptable_t.shape[1]
    qchunk = b // (nw // PROTO_DIM)
    mesh = plsc.VectorSubcoreMesh(core_axis_name="c", subcore_axis_name="s")

    @functools.partial(
        pl.kernel,
        mesh=mesh,
        compiler_params=pltpu.CompilerParams(use_tc_tiling_on_sc=False,
                                             needs_layout_passes=False),
        out_type=jax.ShapeDtypeStruct((PACK_DIM, b), jnp.float32),
        scratch_types=[
            pltpu.VMEM((v,), jnp.float32),
            pltpu.VMEM((pv,), jnp.float32),
            pltpu.VMEM((b,), jnp.int32),
            pltpu.VMEM((qchunk,), jnp.int32),
            pltpu.VMEM((b,), jnp.float32),
            pltpu.VMEM((qchunk,), jnp.float32),
        ],
    )
    def gather(tbl, pidx_hbm, ptbl, qidx_hbm, out,
               row_v, prow_v, pidx_v, qidx_v, pout_v, qout_v):
        wid = lax.axis_index("s") * nc + lax.axis_index("c")
        qf = wid % PROTO_DIM
        qb = (wid // PROTO_DIM) * qchunk
        pltpu.sync_copy(tbl.at[wid], row_v)
        pltpu.sync_copy(ptbl.at[qf], prow_v)
        pltpu.sync_copy(pidx_hbm, pidx_v)
        pltpu.sync_copy(qidx_hbm.at[pl.ds(qb, qchunk)], qidx_v)

        def port_body(i, _):
            vec = pidx_v[pl.ds(i * LANES, LANES)]
            pout_v[pl.ds(i * LANES, LANES)] = plsc.load_gather(row_v, [vec])
            return 0

        lax.fori_loop(0, b // LANES, port_body, 0)

        def proto_body(i, _):
            vec = qidx_v[pl.ds(i * LANES, LANES)]
            qout_v[pl.ds(i * LANES, LANES)] = plsc.load_gather(prow_v, [vec])
            return 0

        lax.fori_loop(0, qchunk // LANES, proto_body, 0)

        pltpu.sync_copy(pout_v, out.at[wid])
        pltpu.sync_copy(qout_v, out.at[PORT_DIM + qf, pl.ds(qb, qchunk)])

    return gather(table_t, pidx, ptable_t, qidx)


def _mlp_body(packed, feat, w1ab, w1c, b1, w2, b2, w3, b3, out):
    h = jnp.dot(feat[...], w1c[...], preferred_element_type=jnp.float32)
    h += lax.dot_general(packed[...], w1ab[...], (((0,), (0,)), ((), ())),
                         preferred_element_type=jnp.float32)
    h = jnp.maximum(h + b1[...], 0.0)
    h = jnp.dot(h, w2[...], preferred_element_type=jnp.float32) + b2[...]
    h = jnp.maximum(h, 0.0)
    out[...] = jnp.dot(h, w3[...], preferred_element_type=jnp.float32) + b3[...]


def _mlp(packed_t, features, w1ab, w1c, b1, w2, b2, w3, b3, *, bm):
    b = features.shape[0]
    n_out = w3.shape[1]
    const = lambda i: (0, 0)
    return pl.pallas_call(
        _mlp_body,
        grid=(b // bm,),
        in_specs=[
            pl.BlockSpec((PACK_DIM, bm), lambda i: (0, i)),
            pl.BlockSpec((bm, features.shape[1]), lambda i: (i, 0)),
            pl.BlockSpec(w1ab.shape, const),
            pl.BlockSpec(w1c.shape, const),
            pl.BlockSpec(b1.shape, const),
            pl.BlockSpec(w2.shape, const),
            pl.BlockSpec(b2.shape, const),
            pl.BlockSpec(w3.shape, const),
            pl.BlockSpec(b3.shape, const),
        ],
        out_specs=pl.BlockSpec((bm, n_out), lambda i: (i, 0)),
        out_shape=jax.ShapeDtypeStruct((b, n_out), jnp.float32),
    )(packed_t, features, w1ab, w1c, b1, w2, b2, w3, b3)


def kernel(port_idx, protocol_idx, features, port_table, proto_table,
           W1, b1, W2, b2, W3, b3):
    b = port_idx.shape[0]
    info = plsc.get_sparse_core_info()
    nc, ns = info.num_cores, info.num_subcores

    packed_t = _sc_gather_t(port_table.T, port_idx.astype(jnp.int32),
                            proto_table.T, protocol_idx.astype(jnp.int32),
                            nc=nc, ns=ns, b=b)

    return _mlp(packed_t, features, W1[:PACK_DIM], W1[PACK_DIM:],
                b1.reshape(1, -1), W2, b2.reshape(1, -1),
                W3, b3.reshape(1, -1), bm=2048)


# speedup vs baseline: 2.4132x; 1.2236x over previous
import functools

import jax
import jax.numpy as jnp
from jax import lax
from jax.experimental import pallas as pl
from jax.experimental.pallas import tpu as pltpu
from jax.experimental.pallas import tpu_sc as plsc

PORT_DIM = 32
PROTO_DIM = 8
PACK_DIM = PORT_DIM + PROTO_DIM
LANES = 16


def _sc_gather_t(table_t, pidx, ptable_t, qidx, *, nc, ns, b):
    nw = nc * ns
    v = table_t.shape[1]
    pv = ptable_t.shape[1]
    qchunk = b // (nw // PROTO_DIM)
    mesh = plsc.VectorSubcoreMesh(core_axis_name="c", subcore_axis_name="s")

    @functools.partial(
        pl.kernel,
        mesh=mesh,
        compiler_params=pltpu.CompilerParams(use_tc_tiling_on_sc=False,
                                             needs_layout_passes=False),
        out_type=jax.ShapeDtypeStruct((PACK_DIM, b), jnp.float32),
        scratch_types=[
            pltpu.VMEM((v,), jnp.float32),
            pltpu.VMEM((pv,), jnp.float32),
            pltpu.VMEM((b,), jnp.int32),
            pltpu.VMEM((qchunk,), jnp.int32),
            pltpu.VMEM((b,), jnp.float32),
            pltpu.VMEM((qchunk,), jnp.float32),
        ],
    )
    def gather(tbl, pidx_hbm, ptbl, qidx_hbm, out,
               row_v, prow_v, pidx_v, qidx_v, pout_v, qout_v):
        wid = lax.axis_index("s") * nc + lax.axis_index("c")
        qf = wid % PROTO_DIM
        qb = (wid // PROTO_DIM) * qchunk
        pltpu.sync_copy(tbl.at[wid], row_v)
        pltpu.sync_copy(ptbl.at[qf], prow_v)
        pltpu.sync_copy(pidx_hbm, pidx_v)
        pltpu.sync_copy(qidx_hbm.at[pl.ds(qb, qchunk)], qidx_v)

        @plsc.parallel_loop(0, b // LANES, unroll=8)
        def port_body(i):
            vec = pidx_v[pl.ds(i * LANES, LANES)]
            pout_v[pl.ds(i * LANES, LANES)] = plsc.load_gather(row_v, [vec])

        @plsc.parallel_loop(0, qchunk // LANES, unroll=8)
        def proto_body(i):
            vec = qidx_v[pl.ds(i * LANES, LANES)]
            qout_v[pl.ds(i * LANES, LANES)] = plsc.load_gather(prow_v, [vec])

        pltpu.sync_copy(pout_v, out.at[wid])
        pltpu.sync_copy(qout_v, out.at[PORT_DIM + qf, pl.ds(qb, qchunk)])

    return gather(table_t, pidx, ptable_t, qidx)


def _mlp_body(packed, feat, w1ab, w1c, b1, w2, b2, w3t, b3, out):
    h = jnp.dot(feat[...], w1c[...], preferred_element_type=jnp.float32)
    h += lax.dot_general(packed[...], w1ab[...], (((0,), (0,)), ((), ())),
                         preferred_element_type=jnp.float32)
    h = jnp.maximum(h + b1[...], 0.0)
    h = jnp.dot(h, w2[...], preferred_element_type=jnp.float32) + b2[...]
    h = jnp.maximum(h, 0.0)
    out[...] = lax.dot_general(w3t[...], h, (((1,), (1,)), ((), ())),
                               preferred_element_type=jnp.float32) + b3[...]


def _mlp(packed_t, features, w1ab, w1c, b1, w2, b2, w3t, b3, *, bm):
    b = features.shape[0]
    n_out = w3t.shape[0]
    const = lambda i: (0, 0)
    return pl.pallas_call(
        _mlp_body,
        grid=(b // bm,),
        in_specs=[
            pl.BlockSpec((PACK_DIM, bm), lambda i: (0, i)),
            pl.BlockSpec((bm, features.shape[1]), lambda i: (i, 0)),
            pl.BlockSpec(w1ab.shape, const),
            pl.BlockSpec(w1c.shape, const),
            pl.BlockSpec(b1.shape, const),
            pl.BlockSpec(w2.shape, const),
            pl.BlockSpec(b2.shape, const),
            pl.BlockSpec(w3t.shape, const),
            pl.BlockSpec(b3.shape, const),
        ],
        out_specs=pl.BlockSpec((n_out, bm), lambda i: (0, i)),
        out_shape=jax.ShapeDtypeStruct((n_out, b), jnp.float32),
    )(packed_t, features, w1ab, w1c, b1, w2, b2, w3t, b3)


def kernel(port_idx, protocol_idx, features, port_table, proto_table,
           W1, b1, W2, b2, W3, b3):
    b = port_idx.shape[0]
    info = plsc.get_sparse_core_info()
    nc, ns = info.num_cores, info.num_subcores

    packed_t = _sc_gather_t(port_table.T, port_idx.astype(jnp.int32),
                            proto_table.T, protocol_idx.astype(jnp.int32),
                            nc=nc, ns=ns, b=b)

    out_t = _mlp(packed_t, features, W1[:PACK_DIM], W1[PACK_DIM:],
                 b1.reshape(1, -1), W2, b2.reshape(1, -1),
                 W3.T, b3.reshape(-1, 1), bm=2048)
    return out_t.T


# speedup vs baseline: 2.4291x; 1.0066x over previous
import functools

import jax
import jax.numpy as jnp
from jax import lax
from jax.experimental import pallas as pl
from jax.experimental.pallas import tpu as pltpu
from jax.experimental.pallas import tpu_sc as plsc

PORT_DIM = 32
PROTO_DIM = 8
PACK_DIM = PORT_DIM + PROTO_DIM
LANES = 16


def _sc_gather_t(table_t, pidx, ptable_t, qidx, *, nc, ns, b):
    nw = nc * ns
    v = table_t.shape[1]
    pv = ptable_t.shape[1]
    qchunk = b // (nw // PROTO_DIM)
    mesh = plsc.VectorSubcoreMesh(core_axis_name="c", subcore_axis_name="s")

    @functools.partial(
        pl.kernel,
        mesh=mesh,
        compiler_params=pltpu.CompilerParams(use_tc_tiling_on_sc=False,
                                             needs_layout_passes=False),
        out_type=jax.ShapeDtypeStruct((PACK_DIM, b), jnp.float32),
        scratch_types=[
            pltpu.VMEM((v,), jnp.float32),
            pltpu.VMEM((pv,), jnp.float32),
            pltpu.VMEM((b,), jnp.int32),
            pltpu.VMEM((qchunk,), jnp.int32),
            pltpu.VMEM((b,), jnp.float32),
            pltpu.VMEM((qchunk,), jnp.float32),
        ],
    )
    def gather(tbl, pidx_hbm, ptbl, qidx_hbm, out,
               row_v, prow_v, pidx_v, qidx_v, pout_v, qout_v):
        wid = lax.axis_index("s") * nc + lax.axis_index("c")
        qf = wid % PROTO_DIM
        qb = (wid // PROTO_DIM) * qchunk
        pltpu.sync_copy(tbl.at[wid], row_v)
        pltpu.sync_copy(ptbl.at[qf], prow_v)
        pltpu.sync_copy(pidx_hbm, pidx_v)
        pltpu.sync_copy(qidx_hbm.at[pl.ds(qb, qchunk)], qidx_v)

        @plsc.parallel_loop(0, b // LANES, unroll=16)
        def port_body(i):
            vec = pidx_v[pl.ds(i * LANES, LANES)]
            pout_v[pl.ds(i * LANES, LANES)] = plsc.load_gather(row_v, [vec])

        @plsc.parallel_loop(0, qchunk // LANES, unroll=16)
        def proto_body(i):
            vec = qidx_v[pl.ds(i * LANES, LANES)]
            qout_v[pl.ds(i * LANES, LANES)] = plsc.load_gather(prow_v, [vec])

        pltpu.sync_copy(pout_v, out.at[wid])
        pltpu.sync_copy(qout_v, out.at[PORT_DIM + qf, pl.ds(qb, qchunk)])

    return gather(table_t, pidx, ptable_t, qidx)


def _mlp_body(packed, feat, w1ab, w1c, b1, w2, b2, w3t, b3, out):
    h = jnp.dot(feat[...], w1c[...], preferred_element_type=jnp.float32)
    h += lax.dot_general(packed[...], w1ab[...], (((0,), (0,)), ((), ())),
                         preferred_element_type=jnp.float32)
    h = jnp.maximum(h + b1[...], 0.0)
    h = jnp.dot(h, w2[...], preferred_element_type=jnp.float32) + b2[...]
    h = jnp.maximum(h, 0.0)
    out[...] = lax.dot_general(w3t[...], h, (((1,), (1,)), ((), ())),
                               preferred_element_type=jnp.float32) + b3[...]


def _mlp(packed_t, features, w1ab, w1c, b1, w2, b2, w3t, b3, *, bm):
    b = features.shape[0]
    n_out = w3t.shape[0]
    const = lambda i: (0, 0)
    return pl.pallas_call(
        _mlp_body,
        grid=(b // bm,),
        in_specs=[
            pl.BlockSpec((PACK_DIM, bm), lambda i: (0, i)),
            pl.BlockSpec((bm, features.shape[1]), lambda i: (i, 0)),
            pl.BlockSpec(w1ab.shape, const),
            pl.BlockSpec(w1c.shape, const),
            pl.BlockSpec(b1.shape, const),
            pl.BlockSpec(w2.shape, const),
            pl.BlockSpec(b2.shape, const),
            pl.BlockSpec(w3t.shape, const),
            pl.BlockSpec(b3.shape, const),
        ],
        out_specs=pl.BlockSpec((n_out, bm), lambda i: (0, i)),
        out_shape=jax.ShapeDtypeStruct((n_out, b), jnp.float32),
    )(packed_t, features, w1ab, w1c, b1, w2, b2, w3t, b3)


def kernel(port_idx, protocol_idx, features, port_table, proto_table,
           W1, b1, W2, b2, W3, b3):
    b = port_idx.shape[0]
    info = plsc.get_sparse_core_info()
    nc, ns = info.num_cores, info.num_subcores

    packed_t = _sc_gather_t(port_table.T, port_idx.astype(jnp.int32),
                            proto_table.T, protocol_idx.astype(jnp.int32),
                            nc=nc, ns=ns, b=b)

    out_t = _mlp(packed_t, features, W1[:PACK_DIM], W1[PACK_DIM:],
                 b1.reshape(1, -1), W2, b2.reshape(1, -1),
                 W3.T, b3.reshape(-1, 1), bm=4096)
    return out_t.T


# speedup vs baseline: 2.9272x; 1.2050x over previous
import functools

import jax
import jax.numpy as jnp
from jax import lax
from jax.experimental import pallas as pl
from jax.experimental.pallas import tpu as pltpu
from jax.experimental.pallas import tpu_sc as plsc

PORT_DIM = 32
PROTO_DIM = 8
PACK_DIM = PORT_DIM + PROTO_DIM
LANES = 16


def _sc_gather_t(table4, pidx, ptable_t, qidx, *, nc, ns, b):
    nw = nc * ns
    vhi, vlo = table4.shape[1], table4.shape[3]
    pv = ptable_t.shape[1]
    qchunk = b // (nw // PROTO_DIM)
    mesh = plsc.VectorSubcoreMesh(core_axis_name="c", subcore_axis_name="s")

    @functools.partial(
        pl.kernel,
        mesh=mesh,
        compiler_params=pltpu.CompilerParams(use_tc_tiling_on_sc=False,
                                             needs_layout_passes=False),
        out_type=jax.ShapeDtypeStruct((PACK_DIM, b), jnp.float32),
        scratch_types=[
            pltpu.VMEM((vhi, vlo), jnp.float32),
            pltpu.VMEM((pv,), jnp.float32),
            pltpu.VMEM((b,), jnp.int32),
            pltpu.VMEM((qchunk,), jnp.int32),
            pltpu.VMEM((b,), jnp.float32),
            pltpu.VMEM((qchunk,), jnp.float32),
        ],
    )
    def gather(tbl, pidx_hbm, ptbl, qidx_hbm, out,
               row_v, prow_v, pidx_v, qidx_v, pout_v, qout_v):
        wid = lax.axis_index("s") * nc + lax.axis_index("c")
        qf = wid % PROTO_DIM
        qb = (wid // PROTO_DIM) * qchunk
        pltpu.sync_copy(tbl.at[wid // 8, :, wid % 8, :], row_v)
        pltpu.sync_copy(ptbl.at[qf], prow_v)
        pltpu.sync_copy(pidx_hbm, pidx_v)
        pltpu.sync_copy(qidx_hbm.at[pl.ds(qb, qchunk)], qidx_v)

        @plsc.parallel_loop(0, b // LANES, unroll=16)
        def port_body(i):
            vec = pidx_v[pl.ds(i * LANES, LANES)]
            pout_v[pl.ds(i * LANES, LANES)] = plsc.load_gather(
                row_v, [lax.shift_right_logical(vec, 7),
                        lax.bitwise_and(vec, 127)])

        @plsc.parallel_loop(0, qchunk // LANES, unroll=16)
        def proto_body(i):
            vec = qidx_v[pl.ds(i * LANES, LANES)]
            qout_v[pl.ds(i * LANES, LANES)] = plsc.load_gather(prow_v, [vec])

        pltpu.sync_copy(pout_v, out.at[wid])
        pltpu.sync_copy(qout_v, out.at[PORT_DIM + qf, pl.ds(qb, qchunk)])

    return gather(table4, pidx, ptable_t, qidx)


def _mlp_body(packed, feat, w1ab, w1c, b1, w2, b2, w3t, b3, out):
    h = jnp.dot(feat[...], w1c[...], preferred_element_type=jnp.float32)
    h += lax.dot_general(packed[...], w1ab[...], (((0,), (0,)), ((), ())),
                         preferred_element_type=jnp.float32)
    h = jnp.maximum(h + b1[...], 0.0)
    h = jnp.dot(h, w2[...], preferred_element_type=jnp.float32) + b2[...]
    h = jnp.maximum(h, 0.0)
    out[...] = lax.dot_general(w3t[...], h, (((1,), (1,)), ((), ())),
                               preferred_element_type=jnp.float32) + b3[...]


def _mlp(packed_t, features, w1ab, w1c, b1, w2, b2, w3t, b3, *, bm):
    b = features.shape[0]
    n_out = w3t.shape[0]
    const = lambda i: (0, 0)
    return pl.pallas_call(
        _mlp_body,
        grid=(b // bm,),
        in_specs=[
            pl.BlockSpec((PACK_DIM, bm), lambda i: (0, i)),
            pl.BlockSpec((bm, features.shape[1]), lambda i: (i, 0)),
            pl.BlockSpec(w1ab.shape, const),
            pl.BlockSpec(w1c.shape, const),
            pl.BlockSpec(b1.shape, const),
            pl.BlockSpec(w2.shape, const),
            pl.BlockSpec(b2.shape, const),
            pl.BlockSpec(w3t.shape, const),
            pl.BlockSpec(b3.shape, const),
        ],
        out_specs=pl.BlockSpec((n_out, bm), lambda i: (0, i)),
        out_shape=jax.ShapeDtypeStruct((n_out, b), jnp.float32),
    )(packed_t, features, w1ab, w1c, b1, w2, b2, w3t, b3)


def kernel(port_idx, protocol_idx, features, port_table, proto_table,
           W1, b1, W2, b2, W3, b3):
    b = port_idx.shape[0]
    info = plsc.get_sparse_core_info()
    nc, ns = info.num_cores, info.num_subcores

    table4 = port_table.T.reshape(4, 8, 512, 128).transpose(0, 2, 1, 3)
    packed_t = _sc_gather_t(table4, port_idx.astype(jnp.int32),
                            proto_table.T, protocol_idx.astype(jnp.int32),
                            nc=nc, ns=ns, b=b)

    out_t = _mlp(packed_t, features, W1[:PACK_DIM], W1[PACK_DIM:],
                 b1.reshape(1, -1), W2, b2.reshape(1, -1),
                 W3.T, b3.reshape(-1, 1), bm=4096)
    return out_t.T


# speedup vs baseline: 3.0149x; 1.0299x over previous
import functools

import jax
import jax.numpy as jnp
from jax import lax
from jax.experimental import pallas as pl
from jax.experimental.pallas import tpu as pltpu
from jax.experimental.pallas import tpu_sc as plsc

PORT_DIM = 32
PROTO_DIM = 8
PACK_DIM = PORT_DIM + PROTO_DIM
LANES = 16


def _sc_gather_t(table4, pidx, ptable_t, qidx, *, nc, ns, b):
    nw = nc * ns
    vhi, vlo = table4.shape[1], table4.shape[3]
    pv = ptable_t.shape[1]
    qchunk = b // (nw // PROTO_DIM)
    mesh = plsc.VectorSubcoreMesh(core_axis_name="c", subcore_axis_name="s")

    @functools.partial(
        pl.kernel,
        mesh=mesh,
        compiler_params=pltpu.CompilerParams(use_tc_tiling_on_sc=False,
                                             needs_layout_passes=False),
        out_type=jax.ShapeDtypeStruct((PACK_DIM, b), jnp.float32),
        scratch_types=[
            pltpu.VMEM((vhi, vlo), jnp.float32),
            pltpu.VMEM((pv,), jnp.float32),
            pltpu.VMEM((b,), jnp.int32),
            pltpu.VMEM((qchunk,), jnp.int32),
            pltpu.VMEM((b,), jnp.float32),
            pltpu.VMEM((qchunk,), jnp.float32),
            pltpu.SemaphoreType.DMA,
        ],
    )
    def gather(tbl, pidx_hbm, ptbl, qidx_hbm, out,
               row_v, prow_v, pidx_v, qidx_v, pout_v, qout_v, sem):
        wid = lax.axis_index("s") * nc + lax.axis_index("c")
        qf = wid % PROTO_DIM
        qb = (wid // PROTO_DIM) * qchunk
        copies = [
            pltpu.async_copy(tbl.at[wid // 8, :, wid % 8, :], row_v, sem),
            pltpu.async_copy(ptbl.at[qf], prow_v, sem),
            pltpu.async_copy(pidx_hbm, pidx_v, sem),
            pltpu.async_copy(qidx_hbm.at[pl.ds(qb, qchunk)], qidx_v, sem),
        ]
        for c in copies:
            c.wait()

        @plsc.parallel_loop(0, b // LANES, unroll=16)
        def port_body(i):
            vec = pidx_v[pl.ds(i * LANES, LANES)]
            pout_v[pl.ds(i * LANES, LANES)] = plsc.load_gather(
                row_v, [lax.shift_right_logical(vec, 7),
                        lax.bitwise_and(vec, 127)])

        @plsc.parallel_loop(0, qchunk // LANES, unroll=16)
        def proto_body(i):
            vec = qidx_v[pl.ds(i * LANES, LANES)]
            qout_v[pl.ds(i * LANES, LANES)] = plsc.load_gather(prow_v, [vec])

        pltpu.sync_copy(pout_v, out.at[wid])
        pltpu.sync_copy(qout_v, out.at[PORT_DIM + qf, pl.ds(qb, qchunk)])

    return gather(table4, pidx, ptable_t, qidx)


def _mlp_body(packed, feat, w1ab, w1c, b1, w2, b2, w3t, b3, out):
    h = jnp.dot(feat[...], w1c[...], preferred_element_type=jnp.float32)
    h += lax.dot_general(packed[...], w1ab[...], (((0,), (0,)), ((), ())),
                         preferred_element_type=jnp.float32)
    h = jnp.maximum(h + b1[...], 0.0)
    h = jnp.dot(h, w2[...], preferred_element_type=jnp.float32) + b2[...]
    h = jnp.maximum(h, 0.0)
    out[...] = lax.dot_general(w3t[...], h, (((1,), (1,)), ((), ())),
                               preferred_element_type=jnp.float32) + b3[...]


def _mlp(packed_t, features, w1ab, w1c, b1, w2, b2, w3t, b3, *, bm):
    b = features.shape[0]
    n_out = w3t.shape[0]
    const = lambda i: (0, 0)
    return pl.pallas_call(
        _mlp_body,
        grid=(b // bm,),
        in_specs=[
            pl.BlockSpec((PACK_DIM, bm), lambda i: (0, i)),
            pl.BlockSpec((bm, features.shape[1]), lambda i: (i, 0)),
            pl.BlockSpec(w1ab.shape, const),
            pl.BlockSpec(w1c.shape, const),
            pl.BlockSpec(b1.shape, const),
            pl.BlockSpec(w2.shape, const),
            pl.BlockSpec(b2.shape, const),
            pl.BlockSpec(w3t.shape, const),
            pl.BlockSpec(b3.shape, const),
        ],
        out_specs=pl.BlockSpec((n_out, bm), lambda i: (0, i)),
        out_shape=jax.ShapeDtypeStruct((n_out, b), jnp.float32),
    )(packed_t, features, w1ab, w1c, b1, w2, b2, w3t, b3)


def kernel(port_idx, protocol_idx, features, port_table, proto_table,
           W1, b1, W2, b2, W3, b3):
    b = port_idx.shape[0]
    info = plsc.get_sparse_core_info()
    nc, ns = info.num_cores, info.num_subcores

    table4 = port_table.T.reshape(4, 8, 512, 128).transpose(0, 2, 1, 3)
    packed_t = _sc_gather_t(table4, port_idx.astype(jnp.int32),
                            proto_table.T, protocol_idx.astype(jnp.int32),
                            nc=nc, ns=ns, b=b)

    out_t = _mlp(packed_t, features, W1[:PACK_DIM], W1[PACK_DIM:],
                 b1.reshape(1, -1), W2, b2.reshape(1, -1),
                 W3.T, b3.reshape(-1, 1), bm=4096)
    return out_t.T
